# Initial kernel scaffold; baseline (speedup 1.0000x reference)
import functools

import jax
import jax.numpy as jnp
from jax import lax
from jax.experimental import pallas as pl
from jax.experimental.pallas import tpu as pltpu
from jax.experimental.pallas import tpu_sc as plsc

N = 10000
D = 128
E = 320000
EPS = 1e-5

NW = 32
EPW = E // NW
B = 128
K = -(-EPW // B)
PAD = K * B - EPW
NP = 10016
RPT = NP // 16

RB = 1000
NBLK = N // RB


_sc_mesh = plsc.VectorSubcoreMesh(core_axis_name="c", subcore_axis_name="s")


@functools.partial(
    pl.kernel,
    out_type=jax.ShapeDtypeStruct((2, NP, D), jnp.float32),
    mesh=_sc_mesh,
    scratch_types=[
        pltpu.VMEM((K, B), jnp.int32),
        pltpu.VMEM((K, B), jnp.int32),
        pltpu.VMEM((B, D), jnp.float32),
        pltpu.VMEM_SHARED((NP, D), jnp.float32),
    ],
)
def _segsum_sc(support_hbm, src_hbm, dst_hbm, zeros_hbm, out_hbm,
               src_v, dst_v, rows_v, acc):
    c = lax.axis_index("c")
    s = lax.axis_index("s")
    w = c * 16 + s
    pltpu.sync_copy(zeros_hbm, acc.at[pl.ds(s * RPT, RPT)])
    pltpu.sync_copy(src_hbm.at[w], src_v)
    pltpu.sync_copy(dst_hbm.at[w], dst_v)
    plsc.subcore_barrier()

    def body(j, carry):
        pltpu.sync_copy(support_hbm.at[src_v.at[j]], rows_v)
        pltpu.sync_copy(rows_v, acc.at[dst_v.at[j]], add=True)
        return carry

    lax.fori_loop(0, K, body, 0)
    plsc.subcore_barrier()
    pltpu.sync_copy(acc.at[pl.ds(s * RPT, RPT)],
                    out_hbm.at[c, pl.ds(s * RPT, RPT)])


def _mm2_body(x_ref, w_ref, ws_ref, b_ref, sup_ref, s_ref):
    x = x_ref[...]
    sup_ref[...] = jnp.dot(x, w_ref[...], preferred_element_type=jnp.float32)
    s_ref[...] = (jnp.dot(x, ws_ref[...], preferred_element_type=jnp.float32)
                  + b_ref[...])


_mm2 = pl.pallas_call(
    _mm2_body,
    grid=(NBLK,),
    in_specs=[
        pl.BlockSpec((RB, D), lambda i: (i, 0)),
        pl.BlockSpec((D, D), lambda i: (0, 0)),
        pl.BlockSpec((D, D), lambda i: (0, 0)),
        pl.BlockSpec((1, D), lambda i: (0, 0)),
    ],
    out_specs=[
        pl.BlockSpec((RB, D), lambda i: (i, 0)),
        pl.BlockSpec((RB, D), lambda i: (i, 0)),
    ],
    out_shape=[
        jax.ShapeDtypeStruct((N, D), jnp.float32),
        jax.ShapeDtypeStruct((N, D), jnp.float32),
    ],
)


def _stats_body(agg_ref, s_ref, t_ref, stats_ref):
    i = pl.program_id(0)
    t = agg_ref[0] + agg_ref[1] + s_ref[...]
    t_ref[...] = t
    su = jnp.sum(t, axis=0, keepdims=True)
    sq = jnp.sum(t * t, axis=0, keepdims=True)
    blk = jnp.concatenate([su, sq, jnp.zeros((6, D), jnp.float32)], axis=0)

    @pl.when(i == 0)
    def _():
        stats_ref[...] = jnp.zeros_like(stats_ref)

    stats_ref[...] += blk


_stats = pl.pallas_call(
    _stats_body,
    grid=(NBLK,),
    in_specs=[
        pl.BlockSpec((2, RB, D), lambda i: (0, i, 0)),
        pl.BlockSpec((RB, D), lambda i: (i, 0)),
    ],
    out_specs=[
        pl.BlockSpec((RB, D), lambda i: (i, 0)),
        pl.BlockSpec((8, D), lambda i: (0, 0)),
    ],
    out_shape=[
        jax.ShapeDtypeStruct((N, D), jnp.float32),
        jax.ShapeDtypeStruct((8, D), jnp.float32),
    ],
)


def _make_norm(has_tot, has_next):
    def body(*refs):
        refs = list(refs)
        t_ref = refs.pop(0)
        stats_ref = refs.pop(0)
        g_ref = refs.pop(0)
        be_ref = refs.pop(0)
        tot_in_ref = refs.pop(0) if has_tot else None
        if has_next:
            wn_ref = refs.pop(0)
            wsn_ref = refs.pop(0)
            bn_ref = refs.pop(0)
        tot_ref = refs.pop(0)
        if has_next:
            sup_ref = refs.pop(0)
            s_out_ref = refs.pop(0)
        stats = stats_ref[...]
        mu = stats[0:1] * (1.0 / N)
        var = stats[1:2] * (1.0 / N) - mu * mu
        inv = lax.rsqrt(var + EPS)
        h = g_ref[...] * (t_ref[...] - mu) * inv + be_ref[...]
        h = jnp.maximum(h, 0.0)
        tot_ref[...] = h + tot_in_ref[...] if has_tot else h
        if has_next:
            sup_ref[...] = jnp.dot(h, wn_ref[...],
                                   preferred_element_type=jnp.float32)
            s_out_ref[...] = (jnp.dot(h, wsn_ref[...],
                                      preferred_element_type=jnp.float32)
                              + bn_ref[...])

    row = pl.BlockSpec((RB, D), lambda i: (i, 0))
    vec = pl.BlockSpec((1, D), lambda i: (0, 0))
    mat = pl.BlockSpec((D, D), lambda i: (0, 0))
    in_specs = [row, pl.BlockSpec((8, D), lambda i: (0, 0)), vec, vec]
    if has_tot:
        in_specs.append(row)
    if has_next:
        in_specs += [mat, mat, vec]
    out_specs = [row]
    out_shape = [jax.ShapeDtypeStruct((N, D), jnp.float32)]
    if has_next:
        out_specs += [row, row]
        out_shape += [jax.ShapeDtypeStruct((N, D), jnp.float32),
                      jax.ShapeDtypeStruct((N, D), jnp.float32)]
    return pl.pallas_call(body, grid=(NBLK,), in_specs=in_specs,
                          out_specs=out_specs, out_shape=out_shape)


_norm_first = _make_norm(has_tot=False, has_next=True)
_norm_mid = _make_norm(has_tot=True, has_next=True)
_norm_last = _make_norm(has_tot=True, has_next=False)


def kernel(x, edge_index, batch, in_W, in_Ws, in_b, in_g, in_be,
           mid_W0, mid_Ws0, mid_b0, mid_g0, mid_be0,
           mid_W1, mid_Ws1, mid_b1, mid_g1, mid_be1):
    del batch
    src = edge_index[0].reshape(NW, EPW)
    dst = edge_index[1].reshape(NW, EPW)
    src3 = jnp.pad(src, ((0, 0), (0, PAD))).reshape(NW, K, B)
    dst3 = jnp.pad(dst, ((0, 0), (0, PAD)), constant_values=N).reshape(NW, K, B)
    zeros = jnp.zeros((RPT, D), jnp.float32)

    def layer(sup, s, g, be, tot, nxt):
        agg2 = _segsum_sc(sup, src3, dst3, zeros)
        t, stats = _stats(agg2, s)
        g = g.reshape(1, D)
        be = be.reshape(1, D)
        if nxt is None:
            return _norm_last(t, stats, g, be, tot)[0]
        wn, wsn, bn = nxt
        bn = bn.reshape(1, D)
        if tot is None:
            return _norm_first(t, stats, g, be, wn, wsn, bn)
        return _norm_mid(t, stats, g, be, tot, wn, wsn, bn)

    sup, s = _mm2(x, in_W, in_Ws, in_b.reshape(1, D))
    tot, sup, s = layer(sup, s, in_g, in_be, None, (mid_W0, mid_Ws0, mid_b0))
    tot, sup, s = layer(sup, s, mid_g0, mid_be0, tot, (mid_W1, mid_Ws1, mid_b1))
    return layer(sup, s, mid_g1, mid_be1, tot, None)


# speedup vs baseline: 4.4456x; 4.4456x over previous
import functools

import jax
import jax.numpy as jnp
from jax import lax
from jax.experimental import pallas as pl
from jax.experimental.pallas import tpu as pltpu
from jax.experimental.pallas import tpu_sc as plsc

N = 10000
D = 128
E = 320000
EPS = 1e-5

NW = 32
EPW = E // NW
B = 128
K = -(-EPW // B)
PAD = K * B - EPW
NP = 10112
RPT = NP // 16

RB = 1000
NBLK = N // RB


_sc_mesh = plsc.VectorSubcoreMesh(core_axis_name="c", subcore_axis_name="s")


@functools.partial(
    pl.kernel,
    out_type=jax.ShapeDtypeStruct((2, NP, D), jnp.float32),
    mesh=_sc_mesh,
    scratch_types=[
        pltpu.VMEM((K, B), jnp.int32),
        pltpu.VMEM((K, B), jnp.int32),
        pltpu.VMEM((B, D), jnp.float32),
        pltpu.VMEM_SHARED((NP, D), jnp.float32),
    ],
)
def _segsum_sc(support_hbm, src_hbm, dst_hbm, zeros_hbm, out_hbm,
               src_v, dst_v, rows_v, acc):
    c = lax.axis_index("c")
    s = lax.axis_index("s")
    w = c * 16 + s
    pltpu.sync_copy(zeros_hbm, acc.at[pl.ds(s * RPT, RPT)])
    pltpu.sync_copy(src_hbm.at[w], src_v)
    pltpu.sync_copy(dst_hbm.at[w], dst_v)
    plsc.subcore_barrier()

    def body(j, carry):
        pltpu.sync_copy(support_hbm.at[src_v.at[j]], rows_v)
        pltpu.sync_copy(rows_v, acc.at[dst_v.at[j]], add=True)
        return carry

    lax.fori_loop(0, K, body, 0)
    plsc.subcore_barrier()
    pltpu.sync_copy(acc.at[pl.ds(s * RPT, RPT)],
                    out_hbm.at[c, pl.ds(s * RPT, RPT)])


def _mm2_body(x_ref, w_ref, ws_ref, b_ref, sup_ref, s_ref):
    x = x_ref[...]
    sup_ref[...] = jnp.dot(x, w_ref[...], preferred_element_type=jnp.float32)
    s_ref[...] = (jnp.dot(x, ws_ref[...], preferred_element_type=jnp.float32)
                  + b_ref[...])


_mm2 = pl.pallas_call(
    _mm2_body,
    grid=(NBLK,),
    in_specs=[
        pl.BlockSpec((RB, D), lambda i: (i, 0)),
        pl.BlockSpec((D, D), lambda i: (0, 0)),
        pl.BlockSpec((D, D), lambda i: (0, 0)),
        pl.BlockSpec((1, D), lambda i: (0, 0)),
    ],
    out_specs=[
        pl.BlockSpec((RB, D), lambda i: (i, 0)),
        pl.BlockSpec((RB, D), lambda i: (i, 0)),
    ],
    out_shape=[
        jax.ShapeDtypeStruct((N, D), jnp.float32),
        jax.ShapeDtypeStruct((N, D), jnp.float32),
    ],
)


def _stats_body(agg_ref, s_ref, t_ref, stats_ref):
    i = pl.program_id(0)
    t = agg_ref[0] + agg_ref[1] + s_ref[...]
    t_ref[...] = t
    su = jnp.sum(t, axis=0, keepdims=True)
    sq = jnp.sum(t * t, axis=0, keepdims=True)
    blk = jnp.concatenate([su, sq, jnp.zeros((6, D), jnp.float32)], axis=0)

    @pl.when(i == 0)
    def _():
        stats_ref[...] = jnp.zeros_like(stats_ref)

    stats_ref[...] += blk


_stats = pl.pallas_call(
    _stats_body,
    grid=(NBLK,),
    in_specs=[
        pl.BlockSpec((2, RB, D), lambda i: (0, i, 0)),
        pl.BlockSpec((RB, D), lambda i: (i, 0)),
    ],
    out_specs=[
        pl.BlockSpec((RB, D), lambda i: (i, 0)),
        pl.BlockSpec((8, D), lambda i: (0, 0)),
    ],
    out_shape=[
        jax.ShapeDtypeStruct((N, D), jnp.float32),
        jax.ShapeDtypeStruct((8, D), jnp.float32),
    ],
)


def _make_norm(has_tot, has_next):
    def body(*refs):
        refs = list(refs)
        t_ref = refs.pop(0)
        stats_ref = refs.pop(0)
        g_ref = refs.pop(0)
        be_ref = refs.pop(0)
        tot_in_ref = refs.pop(0) if has_tot else None
        if has_next:
            wn_ref = refs.pop(0)
            wsn_ref = refs.pop(0)
            bn_ref = refs.pop(0)
        tot_ref = refs.pop(0)
        if has_next:
            sup_ref = refs.pop(0)
            s_out_ref = refs.pop(0)
        stats = stats_ref[...]
        mu = stats[0:1] * (1.0 / N)
        var = stats[1:2] * (1.0 / N) - mu * mu
        inv = lax.rsqrt(var + EPS)
        h = g_ref[...] * (t_ref[...] - mu) * inv + be_ref[...]
        h = jnp.maximum(h, 0.0)
        tot_ref[...] = h + tot_in_ref[...] if has_tot else h
        if has_next:
            sup_ref[...] = jnp.dot(h, wn_ref[...],
                                   preferred_element_type=jnp.float32)
            s_out_ref[...] = (jnp.dot(h, wsn_ref[...],
                                      preferred_element_type=jnp.float32)
                              + bn_ref[...])

    row = pl.BlockSpec((RB, D), lambda i: (i, 0))
    vec = pl.BlockSpec((1, D), lambda i: (0, 0))
    mat = pl.BlockSpec((D, D), lambda i: (0, 0))
    in_specs = [row, pl.BlockSpec((8, D), lambda i: (0, 0)), vec, vec]
    if has_tot:
        in_specs.append(row)
    if has_next:
        in_specs += [mat, mat, vec]
    out_specs = [row]
    out_shape = [jax.ShapeDtypeStruct((N, D), jnp.float32)]
    if has_next:
        out_specs += [row, row]
        out_shape += [jax.ShapeDtypeStruct((N, D), jnp.float32),
                      jax.ShapeDtypeStruct((N, D), jnp.float32)]
    return pl.pallas_call(body, grid=(NBLK,), in_specs=in_specs,
                          out_specs=out_specs, out_shape=out_shape)


_norm_first = _make_norm(has_tot=False, has_next=True)
_norm_mid = _make_norm(has_tot=True, has_next=True)
_norm_last = _make_norm(has_tot=True, has_next=False)


def kernel(x, edge_index, batch, in_W, in_Ws, in_b, in_g, in_be,
           mid_W0, mid_Ws0, mid_b0, mid_g0, mid_be0,
           mid_W1, mid_Ws1, mid_b1, mid_g1, mid_be1):
    del batch
    src = edge_index[0].reshape(NW, EPW)
    dst = edge_index[1].reshape(NW, EPW)
    src3 = jnp.pad(src, ((0, 0), (0, PAD))).reshape(NW, K, B)
    dst3 = jnp.pad(dst, ((0, 0), (0, PAD)), constant_values=N).reshape(NW, K, B)
    zeros = jnp.zeros((RPT, D), jnp.float32)

    def layer(sup, s, g, be, tot, nxt):
        agg2 = _segsum_sc(sup, src3, dst3, zeros)
        t, stats = _stats(agg2, s)
        g = g.reshape(1, D)
        be = be.reshape(1, D)
        if nxt is None:
            return _norm_last(t, stats, g, be, tot)[0]
        wn, wsn, bn = nxt
        bn = bn.reshape(1, D)
        if tot is None:
            return _norm_first(t, stats, g, be, wn, wsn, bn)
        return _norm_mid(t, stats, g, be, tot, wn, wsn, bn)

    sup, s = _mm2(x, in_W, in_Ws, in_b.reshape(1, D))
    tot, sup, s = layer(sup, s, in_g, in_be, None, (mid_W0, mid_Ws0, mid_b0))
    tot, sup, s = layer(sup, s, mid_g0, mid_be0, tot, (mid_W1, mid_Ws1, mid_b1))
    return layer(sup, s, mid_g1, mid_be1, tot, None)
